# Initial kernel scaffold; baseline (speedup 1.0000x reference)
import jax
import jax.numpy as jnp
from jax.experimental import pallas as pl


def kernel(x, spatial, table):
    raise NotImplementedError("write your pallas kernel here")


# speedup vs baseline: 24.3832x; 24.3832x over previous
import functools

import jax
import jax.numpy as jnp
from jax import lax
from jax.experimental import pallas as pl
from jax.experimental.pallas import tpu as pltpu
from jax.experimental.pallas import tpu_sc as plsc

_LANES = 16
_IDX_CHUNK = 8192


def _make_sc_kernel(batch, n, num_spatial, heads):
    nn = n * n
    mesh = plsc.VectorSubcoreMesh(core_axis_name="c", subcore_axis_name="s")

    @functools.partial(
        pl.kernel,
        out_type=jax.ShapeDtypeStruct((batch, heads, nn), jnp.float32),
        mesh=mesh,
        compiler_params=pltpu.CompilerParams(needs_layout_passes=False),
        scratch_types=[
            pltpu.VMEM((num_spatial * heads,), jnp.float32),
            pltpu.VMEM((num_spatial,), jnp.float32),
            pltpu.VMEM((_IDX_CHUNK,), jnp.int32),
            pltpu.VMEM((nn,), jnp.float32),
            pltpu.SemaphoreType.DMA,
        ],
    )
    def sc_kernel(spatial_hbm, table_hbm, out_hbm, table_v, row_v, idx_v,
                  out_v, sem):
        cid = lax.axis_index("c")
        sid = lax.axis_index("s")
        h = sid * 2 + cid

        pltpu.sync_copy(table_hbm, table_v)

        for i in range(num_spatial // _LANES):
            s_idx = lax.iota(jnp.int32, _LANES) + (i * _LANES)
            row_v[pl.ds(i * _LANES, _LANES)] = plsc.load_gather(
                table_v, [s_idx * heads + h])

        for c in range(nn // _IDX_CHUNK):
            pltpu.sync_copy(spatial_hbm.at[pl.ds(c * _IDX_CHUNK, _IDX_CHUNK)],
                            idx_v)

            def body(j, carry, c=c):
                iv = idx_v[pl.ds(j * _LANES, _LANES)]
                out_v[pl.ds(c * _IDX_CHUNK + j * _LANES, _LANES)] = (
                    plsc.load_gather(row_v, [iv]))
                return carry

            lax.fori_loop(0, _IDX_CHUNK // _LANES, body, 0)

        copies = [pltpu.async_copy(out_v, out_hbm.at[b, h], sem)
                  for b in range(batch)]
        for cp in copies:
            cp.wait()

    return sc_kernel


def kernel(x, spatial, table):
    batch = x.shape[0]
    n = spatial.shape[0]
    num_spatial, heads = table.shape
    sp_flat = spatial.reshape(-1).astype(jnp.int32)
    tab_flat = table.reshape(-1)
    out = _make_sc_kernel(batch, n, num_spatial, heads)(sp_flat, tab_flat)
    return out.reshape(batch, heads, n, n)


# speedup vs baseline: 25.4931x; 1.0455x over previous
import functools

import jax
import jax.numpy as jnp
from jax import lax
from jax.experimental import pallas as pl
from jax.experimental.pallas import tpu as pltpu
from jax.experimental.pallas import tpu_sc as plsc

_LANES = 16
_IDX_CHUNK = 8192


def _make_sc_kernel(batch, n, num_spatial, heads):
    nn = n * n
    mesh = plsc.VectorSubcoreMesh(core_axis_name="c", subcore_axis_name="s")

    @functools.partial(
        pl.kernel,
        out_type=jax.ShapeDtypeStruct((batch, heads, nn), jnp.float32),
        mesh=mesh,
        compiler_params=pltpu.CompilerParams(needs_layout_passes=False),
        scratch_types=[
            pltpu.VMEM((num_spatial * heads,), jnp.float32),
            pltpu.VMEM((num_spatial,), jnp.float32),
            pltpu.VMEM((_IDX_CHUNK,), jnp.int32),
            pltpu.VMEM((nn,), jnp.float32),
            pltpu.SemaphoreType.DMA,
        ],
    )
    def sc_kernel(spatial_hbm, table_hbm, out_hbm, table_v, row_v, idx_v,
                  out_v, sem):
        cid = lax.axis_index("c")
        sid = lax.axis_index("s")
        h = sid * 2 + cid

        pltpu.sync_copy(table_hbm, table_v)

        for i in range(num_spatial // _LANES):
            s_idx = lax.iota(jnp.int32, _LANES) + (i * _LANES)
            row_v[pl.ds(i * _LANES, _LANES)] = plsc.load_gather(
                table_v, [s_idx * heads + h])

        unroll = 8
        step = _LANES * unroll
        pending = []
        for c in range(nn // _IDX_CHUNK):
            pltpu.sync_copy(spatial_hbm.at[pl.ds(c * _IDX_CHUNK, _IDX_CHUNK)],
                            idx_v)

            def body(j, carry, c=c):
                base = j * step
                for u in range(unroll):
                    off = base + u * _LANES
                    iv = idx_v[pl.ds(off, _LANES)]
                    out_v[pl.ds(c * _IDX_CHUNK + off, _LANES)] = (
                        plsc.load_gather(row_v, [iv]))
                return carry

            lax.fori_loop(0, _IDX_CHUNK // step, body, 0)

            if len(pending) == 2:
                for cp in pending.pop(0):
                    cp.wait()
            pending.append([
                pltpu.async_copy(
                    out_v.at[pl.ds(c * _IDX_CHUNK, _IDX_CHUNK)],
                    out_hbm.at[b, h, pl.ds(c * _IDX_CHUNK, _IDX_CHUNK)],
                    sem)
                for b in range(batch)])
        for chunk_copies in pending:
            for cp in chunk_copies:
                cp.wait()

    return sc_kernel


def kernel(x, spatial, table):
    batch = x.shape[0]
    n = spatial.shape[0]
    num_spatial, heads = table.shape
    sp_flat = spatial.reshape(-1).astype(jnp.int32)
    tab_flat = table.reshape(-1)
    out = _make_sc_kernel(batch, n, num_spatial, heads)(sp_flat, tab_flat)
    return out.reshape(batch, heads, n, n)


# speedup vs baseline: 60.0551x; 2.3557x over previous
import functools

import jax
import jax.numpy as jnp
from jax import lax
from jax.experimental import pallas as pl
from jax.experimental.pallas import tpu as pltpu
from jax.experimental.pallas import tpu_sc as plsc

_LANES = 16
_IDX_CHUNK = 8192


def _make_sc_gather(n, num_spatial, heads):
    nn = n * n
    mesh = plsc.VectorSubcoreMesh(core_axis_name="c", subcore_axis_name="s")

    @functools.partial(
        pl.kernel,
        out_type=jax.ShapeDtypeStruct((heads, n, n), jnp.float32),
        mesh=mesh,
        compiler_params=pltpu.CompilerParams(needs_layout_passes=False),
        scratch_types=[
            pltpu.VMEM((num_spatial * heads,), jnp.float32),
            pltpu.VMEM((num_spatial,), jnp.float32),
            pltpu.VMEM((2, _IDX_CHUNK), jnp.int32),
            pltpu.VMEM((n, n), jnp.float32),
            pltpu.SemaphoreType.DMA,
            pltpu.SemaphoreType.DMA,
        ],
    )
    def sc_gather(spatial_hbm, table_hbm, out_hbm, table_v, row_v, idx_v,
                  out_v, sem, idx_sem):
        cid = lax.axis_index("c")
        sid = lax.axis_index("s")
        h = sid * 2 + cid

        n_chunks = nn // _IDX_CHUNK
        rows_per_chunk = _IDX_CHUNK // n

        def idx_fetch(c):
            return pltpu.async_copy(
                spatial_hbm.at[pl.ds(c * _IDX_CHUNK, _IDX_CHUNK)],
                idx_v.at[c % 2], idx_sem)

        idx_pending = idx_fetch(0)
        pltpu.sync_copy(table_hbm, table_v)

        for i in range(num_spatial // _LANES):
            s_idx = lax.iota(jnp.int32, _LANES) + (i * _LANES)
            row_v[pl.ds(i * _LANES, _LANES)] = plsc.load_gather(
                table_v, [s_idx * heads + h])

        pending = []
        for c in range(n_chunks):
            idx_pending.wait()
            if c + 1 < n_chunks:
                idx_pending = idx_fetch(c + 1)
            buf = c % 2

            def row_body(r, c=c, buf=buf):
                row = c * rows_per_chunk + r
                for u in range(n // _LANES):
                    iv = idx_v[buf, pl.ds(r * n + u * _LANES, _LANES)]
                    out_v[row, pl.ds(u * _LANES, _LANES)] = (
                        plsc.load_gather(row_v, [iv]))

            plsc.parallel_loop(0, rows_per_chunk, unroll=8)(row_body)

            if len(pending) == 2:
                pending.pop(0).wait()
            pending.append(
                pltpu.async_copy(
                    out_v.at[pl.ds(c * rows_per_chunk, rows_per_chunk)],
                    out_hbm.at[h, pl.ds(c * rows_per_chunk, rows_per_chunk)],
                    sem))
        for cp in pending:
            cp.wait()

    return sc_gather


def _make_tc_broadcast(batch, n, heads):
    def body(bias_ref, out_ref, sem):
        copies = [pltpu.async_copy(bias_ref, out_ref.at[b], sem)
                  for b in range(batch)]
        for cp in copies:
            cp.wait()

    return pl.pallas_call(
        body,
        in_specs=[pl.BlockSpec(memory_space=pltpu.VMEM)],
        out_specs=pl.BlockSpec(memory_space=pl.ANY),
        out_shape=jax.ShapeDtypeStruct((batch, heads, n, n), jnp.float32),
        scratch_shapes=[pltpu.SemaphoreType.DMA],
    )


def kernel(x, spatial, table):
    batch = x.shape[0]
    n = spatial.shape[0]
    num_spatial, heads = table.shape
    sp_flat = spatial.reshape(-1).astype(jnp.int32)
    tab_flat = table.reshape(-1)
    bias = _make_sc_gather(n, num_spatial, heads)(sp_flat, tab_flat)
    return _make_tc_broadcast(batch, n, heads)(bias)


# speedup vs baseline: 64.7646x; 1.0784x over previous
import functools

import jax
import jax.numpy as jnp
from jax import lax
from jax.experimental import pallas as pl
from jax.experimental.pallas import tpu as pltpu
from jax.experimental.pallas import tpu_sc as plsc

_LANES = 16
_IDX_CHUNK = 16384


def _make_sc_gather(n, num_spatial, heads):
    nn = n * n
    mesh = plsc.VectorSubcoreMesh(core_axis_name="c", subcore_axis_name="s")

    @functools.partial(
        pl.kernel,
        out_type=jax.ShapeDtypeStruct((heads, n, n), jnp.float32),
        mesh=mesh,
        compiler_params=pltpu.CompilerParams(needs_layout_passes=False),
        scratch_types=[
            pltpu.VMEM((num_spatial * heads,), jnp.float32),
            pltpu.VMEM((num_spatial,), jnp.float32),
            pltpu.VMEM((2, _IDX_CHUNK), jnp.int32),
            pltpu.VMEM((n, n), jnp.float32),
            pltpu.SemaphoreType.DMA,
            pltpu.SemaphoreType.DMA,
        ],
    )
    def sc_gather(spatial_hbm, table_hbm, out_hbm, table_v, row_v, idx_v,
                  out_v, sem, idx_sem):
        cid = lax.axis_index("c")
        sid = lax.axis_index("s")
        h = sid * 2 + cid

        n_chunks = nn // _IDX_CHUNK
        rows_per_chunk = _IDX_CHUNK // n

        def idx_fetch(c):
            return pltpu.async_copy(
                spatial_hbm.at[pl.ds(c * _IDX_CHUNK, _IDX_CHUNK)],
                idx_v.at[c % 2], idx_sem)

        idx_pending = idx_fetch(0)
        pltpu.sync_copy(table_hbm, table_v)

        for i in range(num_spatial // _LANES):
            s_idx = lax.iota(jnp.int32, _LANES) + (i * _LANES)
            row_v[pl.ds(i * _LANES, _LANES)] = plsc.load_gather(
                table_v, [s_idx * heads + h])

        pending = []
        for c in range(n_chunks):
            idx_pending.wait()
            if c + 1 < n_chunks:
                idx_pending = idx_fetch(c + 1)
            buf = c % 2

            def row_body(r, c=c, buf=buf):
                row = c * rows_per_chunk + r
                for u in range(n // _LANES):
                    iv = idx_v[buf, pl.ds(r * n + u * _LANES, _LANES)]
                    out_v[row, pl.ds(u * _LANES, _LANES)] = (
                        plsc.load_gather(row_v, [iv]))

            plsc.parallel_loop(0, rows_per_chunk, unroll=4)(row_body)

            if len(pending) == 2:
                pending.pop(0).wait()
            pending.append(
                pltpu.async_copy(
                    out_v.at[pl.ds(c * rows_per_chunk, rows_per_chunk)],
                    out_hbm.at[h, pl.ds(c * rows_per_chunk, rows_per_chunk)],
                    sem))
        for cp in pending:
            cp.wait()

    return sc_gather


def _make_tc_broadcast(batch, n, heads):
    def body(bias_ref, out_ref, sem):
        copies = [pltpu.async_copy(bias_ref, out_ref.at[b], sem)
                  for b in range(batch)]
        for cp in copies:
            cp.wait()

    return pl.pallas_call(
        body,
        in_specs=[pl.BlockSpec(memory_space=pltpu.VMEM)],
        out_specs=pl.BlockSpec(memory_space=pl.ANY),
        out_shape=jax.ShapeDtypeStruct((batch, heads, n, n), jnp.float32),
        scratch_shapes=[pltpu.SemaphoreType.DMA],
    )


def kernel(x, spatial, table):
    batch = x.shape[0]
    n = spatial.shape[0]
    num_spatial, heads = table.shape
    sp_flat = spatial.reshape(-1).astype(jnp.int32)
    tab_flat = table.reshape(-1)
    bias = _make_sc_gather(n, num_spatial, heads)(sp_flat, tab_flat)
    return _make_tc_broadcast(batch, n, heads)(bias)
